# baseline (speedup 1.0000x reference)
import functools

import jax
import jax.numpy as jnp
from jax import lax
from jax.experimental import pallas as pl
from jax.experimental.pallas import tpu as pltpu
from jax.experimental.pallas import tpu_sc as plsc

_NUM_CORES = 2
_NUM_SUBCORES = 16
_NUM_WORKERS = _NUM_CORES * _NUM_SUBCORES
_CHUNK = 16
_SLOTS = 3


def _gather_call(batch, seq, hidden, idx, table):
    total = batch * seq
    b_per_w = total // _NUM_WORKERS
    w_per_row = seq // b_per_w
    n_chunks = b_per_w // _CHUNK
    n_rounds = n_chunks + 2
    n_rounds += (-n_rounds) % _SLOTS
    mesh = plsc.VectorSubcoreMesh(core_axis_name="c", subcore_axis_name="s")

    @functools.partial(
        pl.kernel,
        mesh=mesh,
        out_type=jax.ShapeDtypeStruct((batch, seq, hidden), jnp.float32),
        scratch_types=[
            pltpu.VMEM((b_per_w,), jnp.int32),
            pltpu.VMEM_SHARED(
                (_NUM_SUBCORES, _SLOTS * _CHUNK, hidden), jnp.float32
            ),
            pltpu.SemaphoreType.DMA,
            pltpu.SemaphoreType.DMA,
            pltpu.SemaphoreType.DMA,
            pltpu.SemaphoreType.DMA,
        ],
    )
    def _gather(idx_hbm, table_hbm, out_hbm, idx_v, ring, s0, s1, s2, s_out):
        cid = lax.axis_index("c")
        sid = lax.axis_index("s")
        wid = sid * _NUM_CORES + cid
        row = wid // w_per_row
        off = (wid % w_per_row) * b_per_w
        pltpu.sync_copy(idx_hbm.at[row, pl.ds(off, b_per_w)], idx_v)
        sg = (s0, s1, s2)

        def round_(r, p):
            @pl.when((r >= _SLOTS) & (r < n_chunks + _SLOTS))
            def _():
                pltpu.make_async_copy(
                    ring.at[sid, pl.ds(0, _CHUNK)],
                    out_hbm.at[row, pl.ds(off, _CHUNK)],
                    s_out,
                ).wait()

            @pl.when((r >= 2) & (r < n_chunks + 2))
            def _():
                c = r - 2
                pltpu.make_async_copy(
                    table_hbm.at[pl.ds(0, _CHUNK)],
                    ring.at[sid, pl.ds(0, _CHUNK)],
                    sg[(p - 2) % _SLOTS],
                ).wait()
                pltpu.async_copy(
                    ring.at[sid, pl.ds(((p - 2) % _SLOTS) * _CHUNK, _CHUNK)],
                    out_hbm.at[row, pl.ds(off + c * _CHUNK, _CHUNK)],
                    s_out,
                )

            @pl.when(r < n_chunks)
            def _():
                vec = idx_v[pl.ds(r * _CHUNK, _CHUNK)]
                for j in range(_CHUNK):
                    pltpu.async_copy(
                        table_hbm.at[vec[j]],
                        ring.at[sid, p * _CHUNK + j],
                        sg[p],
                    )

        def body(i, carry):
            for p in range(_SLOTS):
                round_(i * _SLOTS + p, p)
            return carry

        lax.fori_loop(0, n_rounds // _SLOTS, body, 0)

    return _gather(idx, table)


def kernel(position_ids, weight):
    batch, seq = position_ids.shape
    vocab, hidden = weight.shape
    idx = position_ids.astype(jnp.int32)
    return _gather_call(batch, seq, hidden, idx, weight)
